# Initial kernel scaffold; baseline (speedup 1.0000x reference)
import jax
import jax.numpy as jnp
from jax.experimental import pallas as pl


def kernel(features, item_table, rating_table, W_r1, b_r1, A1, bA1, A2, bA2, G1, bG1, G2, bG2, W1, b1, nodes, history_u, history_r):
    raise NotImplementedError("write your pallas kernel here")


# speedup vs baseline: 4.0635x; 4.0635x over previous
import functools

import jax
import jax.numpy as jnp
from jax import lax
from jax.experimental import pallas as pl
from jax.experimental.pallas import tpu as pltpu
from jax.experimental.pallas import tpu_sc as plsc

N, G, L, D = 10000, 8, 16, 128
BN = 400
GATHER_W = 128

_INTERPRET = False


def _sc_gather_rows(table, idx):
    M = idx.shape[1]
    d = table.shape[1]
    mesh = plsc.VectorSubcoreMesh(core_axis_name="core",
                                  subcore_axis_name="subcore")

    @functools.partial(
        pl.kernel,
        out_type=jax.ShapeDtypeStruct((M, d), table.dtype),
        mesh=mesh,
    )
    def gath(x_hbm, i_hbm, o_hbm):
        def body(i_vmem, o_vmem):
            pltpu.sync_copy(x_hbm.at[i_vmem.at[0]], o_vmem)

        pltpu.emit_pipeline(
            body,
            grid=(M // GATHER_W,),
            in_specs=[pl.BlockSpec((1, GATHER_W), lambda i: (0, i))],
            out_specs=[pl.BlockSpec((GATHER_W, d), lambda i: (i, 0))],
            core_axis_name=("core", "subcore"),
            dimension_semantics=(pltpu.PARALLEL,),
        )(i_hbm, o_hbm)

    return gath(table, idx)


def _tc_body(mem_ref, ei_ref, hr_ref, rt_ref, wr1_ref, br1_ref,
             a1_ref, ba1_ref, a2_ref, ba2_ref,
             g1_ref, bg1_ref, g2_ref, bg2_ref,
             w1_ref, b1_ref, out_ref):
    f32 = jnp.float32
    members = mem_ref[...]
    m3 = members.reshape(BN, G, D)
    u = jnp.mean(m3, axis=1)

    h = jnp.maximum(
        jnp.dot(members, g1_ref[...], preferred_element_type=f32)
        + bg1_ref[...], 0.0)
    gs = jnp.dot(h, g2_ref[...], preferred_element_type=f32) + ba_scalar(bg2_ref)
    gs3 = gs.reshape(BN, G, 1)
    gmax = jnp.max(gs3, axis=1, keepdims=True)
    ge = jnp.exp(gs3 - gmax)
    gatt = ge / jnp.sum(ge, axis=1, keepdims=True)
    self_feats = jnp.sum(gatt * m3, axis=1)

    ei = ei_ref[...]
    hr = hr_ref[...]
    onehot = (hr == lax.broadcasted_iota(jnp.int32, (BN * L, 5), 1)).astype(f32)
    rt = rt_ref[...]
    er = jnp.dot(onehot, rt, preferred_element_type=f32)
    ut = jnp.broadcast_to(u.reshape(BN, 1, D), (BN, L, D)).reshape(BN * L, D)

    rtWb = jnp.dot(rt, wr1_ref[D:2 * D, :], preferred_element_type=f32)
    t = jnp.dot(ei, wr1_ref[0:D, :], preferred_element_type=f32)
    t = t + jnp.dot(onehot, rtWb, preferred_element_type=f32)
    t = t + jnp.dot(ei * er, wr1_ref[3 * D:4 * D, :], preferred_element_type=f32)
    t = t + jnp.dot(ei * ut, wr1_ref[4 * D:5 * D, :], preferred_element_type=f32)
    t = t + jnp.dot(er * ut, wr1_ref[5 * D:6 * D, :], preferred_element_type=f32)
    tu = jnp.dot(u, wr1_ref[2 * D:3 * D, :], preferred_element_type=f32)
    x3 = t.reshape(BN, L, D) + tu.reshape(BN, 1, D) + br1_ref[...].reshape(1, 1, D)
    x3 = jnp.maximum(x3, 0.0)
    x2 = x3.reshape(BN * L, D)

    a = jnp.maximum(
        jnp.dot(x2, a1_ref[...], preferred_element_type=f32) + ba1_ref[...], 0.0)
    s = jnp.dot(a, a2_ref[...], preferred_element_type=f32) + ba_scalar(ba2_ref)
    s3 = s.reshape(BN, L, 1)
    smax = jnp.max(s3, axis=1, keepdims=True)
    se = jnp.exp(s3 - smax)
    att = se / jnp.sum(se, axis=1, keepdims=True)
    neigh = jnp.sum(att * x3, axis=1)

    o = jnp.dot(self_feats, w1_ref[0:D, :], preferred_element_type=f32)
    o = o + jnp.dot(neigh, w1_ref[D:2 * D, :], preferred_element_type=f32)
    out_ref[...] = jnp.maximum(o + b1_ref[...], 0.0)


def ba_scalar(ref):
    return ref[0, 0]


def _tc_forward(members_flat, ei_flat, hr_col, rating_table,
                W_r1, b_r1, A1, bA1, A2, bA2, G1, bG1, G2, bG2, W1, b1):
    grid = (N // BN,)
    const = lambda i: (0, 0)
    return pl.pallas_call(
        _tc_body,
        grid=grid,
        in_specs=[
            pl.BlockSpec((BN * G, D), lambda i: (i, 0)),
            pl.BlockSpec((BN * L, D), lambda i: (i, 0)),
            pl.BlockSpec((BN * L, 1), lambda i: (i, 0)),
            pl.BlockSpec((5, D), const),
            pl.BlockSpec((6 * D, D), const),
            pl.BlockSpec((1, D), const),
            pl.BlockSpec((D, 16), const),
            pl.BlockSpec((1, 16), const),
            pl.BlockSpec((16, 1), const),
            pl.BlockSpec((1, 1), const),
            pl.BlockSpec((D, 16), const),
            pl.BlockSpec((1, 16), const),
            pl.BlockSpec((16, 1), const),
            pl.BlockSpec((1, 1), const),
            pl.BlockSpec((2 * D, D), const),
            pl.BlockSpec((1, D), const),
        ],
        out_specs=pl.BlockSpec((BN, D), lambda i: (i, 0)),
        out_shape=jax.ShapeDtypeStruct((N, D), jnp.float32),
        interpret=_INTERPRET,
    )(members_flat, ei_flat, hr_col, rating_table,
      W_r1, b_r1.reshape(1, D), A1, bA1.reshape(1, 16), A2, bA2.reshape(1, 1),
      G1, bG1.reshape(1, 16), G2, bG2.reshape(1, 1), W1, b1.reshape(1, D))


def kernel(features, item_table, rating_table, W_r1, b_r1, A1, bA1, A2, bA2,
           G1, bG1, G2, bG2, W1, b1, nodes, history_u, history_r):
    nodes_i = nodes.astype(jnp.int32).reshape(1, N * G)
    hist_i = history_u.astype(jnp.int32).reshape(1, N * L)
    members_flat = _sc_gather_rows(features, nodes_i)
    ei_flat = _sc_gather_rows(item_table, hist_i)
    hr_col = history_r.astype(jnp.int32).reshape(N * L, 1)
    return _tc_forward(members_flat, ei_flat, hr_col, rating_table,
                       W_r1, b_r1, A1, bA1, A2, bA2, G1, bG1, G2, bG2, W1, b1)
